# baseline (speedup 1.0000x reference)
import functools

import jax
import jax.numpy as jnp
from jax import lax
from jax.experimental import pallas as pl
from jax.experimental.pallas import tpu as pltpu
from jax.experimental.pallas import tpu_sc as plsc

_NBUF = 4


@functools.cache
def _make_gather(batch, seq, V, D, nbuf):
    info = plsc.get_sparse_core_info()
    num_workers = info.num_cores * info.num_subcores
    rows_per_w = batch // num_workers
    n_groups = rows_per_w // nbuf
    assert rows_per_w * num_workers == batch and n_groups * nbuf == rows_per_w

    mesh = plsc.VectorSubcoreMesh(core_axis_name="c", subcore_axis_name="s")

    @functools.partial(
        pl.kernel,
        mesh=mesh,
        out_type=jax.ShapeDtypeStruct((batch, seq, D), jnp.float32),
        scratch_types=[
            pltpu.VMEM((rows_per_w, seq), jnp.int32),
            *[pltpu.VMEM((seq, D), jnp.float32) for _ in range(nbuf)],
            *[pltpu.SemaphoreType.DMA for _ in range(nbuf)],
            pltpu.SemaphoreType.DMA,
        ],
        compiler_params=pltpu.CompilerParams(use_tc_tiling_on_sc=False),
    )
    def gather_kernel(idx_hbm, table_hbm, out_hbm, idx_v, *bufs):
        rows = bufs[:nbuf]
        gsem = bufs[nbuf : 2 * nbuf]
        ssem = bufs[2 * nbuf]
        wid = lax.axis_index("s") * info.num_cores + lax.axis_index("c")
        base = wid * rows_per_w

        pltpu.sync_copy(idx_hbm.at[pl.ds(base, rows_per_w)], idx_v)

        def gather_start(b, c):
            pltpu.async_copy(table_hbm.at[idx_v.at[c]], rows[b], gsem[b])

        def gather_wait(b):
            pltpu.make_async_copy(table_hbm.at[idx_v.at[0]], rows[b], gsem[b]).wait()

        for b in range(nbuf):
            gather_start(b, b)

        def body(g, carry):
            for b in range(nbuf):
                c = g * nbuf + b
                gather_wait(b)
                copy = pltpu.make_async_copy(rows[b], out_hbm.at[base + c], ssem)
                copy.start()
                copy.wait()

                @pl.when(c + nbuf < rows_per_w)
                def _():
                    gather_start(b, c + nbuf)

            return carry

        lax.fori_loop(0, n_groups, body, 0)

    return gather_kernel


def kernel(card_indices, table):
    batch, seq = card_indices.shape
    vocab, dim = table.shape
    gather = _make_gather(batch, seq, vocab, dim, _NBUF)
    return gather(card_indices.astype(jnp.int32), table)


# speedup vs baseline: 1.4311x; 1.4311x over previous
import functools

import jax
import jax.numpy as jnp
from jax import lax
from jax.experimental import pallas as pl
from jax.experimental.pallas import tpu as pltpu
from jax.experimental.pallas import tpu_sc as plsc


@functools.cache
def _make_gather(batch, seq, V, D):
    info = plsc.get_sparse_core_info()
    L = info.num_lanes
    num_workers = info.num_cores * info.num_subcores
    n_rounds = D // num_workers
    unroll = 8
    n_groups = batch // (L * unroll)
    assert n_rounds * num_workers == D
    assert n_groups * L * unroll == batch and seq % 2 == 0

    mesh = plsc.VectorSubcoreMesh(core_axis_name="c", subcore_axis_name="s")

    @functools.partial(
        pl.kernel,
        mesh=mesh,
        out_type=jax.ShapeDtypeStruct((seq, D, batch), jnp.float32),
        scratch_types=[
            pltpu.VMEM((1, V), jnp.float32),
            *[pltpu.VMEM((1, batch), jnp.int32) for _ in range(2)],
            *[pltpu.VMEM((1, batch), jnp.float32) for _ in range(2)],
            *[pltpu.SemaphoreType.DMA for _ in range(2)],
            *[pltpu.SemaphoreType.DMA for _ in range(2)],
        ],
        compiler_params=pltpu.CompilerParams(
            use_tc_tiling_on_sc=True, needs_layout_passes=False
        ),
    )
    def gather_kernel(idx_hbm, table_hbm, out_hbm, trow, i0, i1, o0, o1, si0, si1, so0, so1):
        ibuf, obuf = (i0, i1), (o0, o1)
        isem, osem = (si0, si1), (so0, so1)
        wid = lax.axis_index("s") * info.num_cores + lax.axis_index("c")
        z16 = jnp.zeros((L,), jnp.int32)

        def idx_load(p, s):
            pltpu.async_copy(idx_hbm.at[pl.ds(s, 1)], ibuf[p], isem[p])

        def idx_wait(p):
            pltpu.make_async_copy(idx_hbm.at[pl.ds(0, 1)], ibuf[p], isem[p]).wait()

        def out_store(p, s, d):
            pltpu.async_copy(obuf[p], out_hbm.at[s, pl.ds(d, 1)], osem[p])

        def out_wait(p):
            pltpu.make_async_copy(obuf[p], out_hbm.at[0, pl.ds(0, 1)], osem[p]).wait()

        def run_round(r, carry):
            d = r * num_workers + wid
            pltpu.sync_copy(table_hbm.at[pl.ds(d, 1)], trow)
            idx_load(0, 0)
            idx_load(1, 1)

            def do_row(p, s, first_pair):
                idx_wait(p)

                @pl.when(jnp.logical_not(first_pair))
                def _():
                    out_wait(p)

                def inner(i, c):
                    for u in range(unroll):
                        off = (i * unroll + u) * L
                        v = ibuf[p][0, pl.ds(off, L)]
                        obuf[p][0, pl.ds(off, L)] = plsc.load_gather(trow, [z16, v])
                    return c

                lax.fori_loop(0, n_groups, inner, 0)
                out_store(p, s, d)

                @pl.when(s + 2 < seq)
                def _():
                    idx_load(p, s + 2)

            def pair(g, c):
                do_row(0, 2 * g, g == 0)
                do_row(1, 2 * g + 1, g == 0)
                return c

            lax.fori_loop(0, seq // 2, pair, 0)
            out_wait(0)
            out_wait(1)
            return carry

        lax.fori_loop(0, n_rounds, run_round, 0)

    return gather_kernel


def kernel(card_indices, table):
    batch, seq = card_indices.shape
    vocab, dim = table.shape
    idx_t = card_indices.astype(jnp.int32).T
    table_t = table.T
    gather = _make_gather(batch, seq, vocab, dim)
    out_t = gather(idx_t, table_t)
    return jnp.transpose(out_t, (2, 0, 1))


# speedup vs baseline: 2.1155x; 1.4782x over previous
import functools

import jax
import jax.numpy as jnp
from jax import lax
from jax.experimental import pallas as pl
from jax.experimental.pallas import tpu as pltpu
from jax.experimental.pallas import tpu_sc as plsc


@functools.cache
def _make_gather(batch, seq, V, D):
    info = plsc.get_sparse_core_info()
    L = info.num_lanes
    num_workers = info.num_cores * info.num_subcores
    n_rounds = D // num_workers
    unroll = 8
    n_groups = batch // (L * unroll)
    assert n_rounds * num_workers == D
    assert n_groups * L * unroll == batch and seq % 2 == 0

    mesh = plsc.VectorSubcoreMesh(core_axis_name="c", subcore_axis_name="s")

    @functools.partial(
        pl.kernel,
        mesh=mesh,
        out_type=jax.ShapeDtypeStruct((seq, D, batch), jnp.float32),
        scratch_types=[
            pltpu.VMEM((1, V), jnp.float32),
            *[pltpu.VMEM((1, batch), jnp.int32) for _ in range(2)],
            *[pltpu.VMEM((1, batch), jnp.float32) for _ in range(2)],
            *[pltpu.SemaphoreType.DMA for _ in range(2)],
            *[pltpu.SemaphoreType.DMA for _ in range(2)],
        ],
        compiler_params=pltpu.CompilerParams(
            use_tc_tiling_on_sc=True, needs_layout_passes=False
        ),
    )
    def gather_kernel(idx_hbm, table_hbm, out_hbm, trow, i0, i1, o0, o1, si0, si1, so0, so1):
        ibuf, obuf = (i0, i1), (o0, o1)
        isem, osem = (si0, si1), (so0, so1)
        wid = lax.axis_index("s") * info.num_cores + lax.axis_index("c")
        z16 = jnp.zeros((L,), jnp.int32)

        def idx_load(p, s):
            pltpu.async_copy(idx_hbm.at[pl.ds(s, 1)], ibuf[p], isem[p])

        def idx_wait(p):
            pltpu.make_async_copy(idx_hbm.at[pl.ds(0, 1)], ibuf[p], isem[p]).wait()

        def out_store(p, s, d):
            pltpu.async_copy(obuf[p], out_hbm.at[s, pl.ds(d, 1)], osem[p])

        def out_wait(p):
            pltpu.make_async_copy(obuf[p], out_hbm.at[0, pl.ds(0, 1)], osem[p]).wait()

        def run_round(r, carry):
            d = r * num_workers + wid
            pltpu.sync_copy(table_hbm.at[pl.ds(d, 1)], trow)
            idx_load(0, 0)
            idx_load(1, 1)

            def do_row(p, s, first_pair):
                idx_wait(p)

                @pl.when(jnp.logical_not(first_pair))
                def _():
                    out_wait(p)

                @plsc.parallel_loop(0, batch, L, unroll=unroll)
                def _inner(off):
                    v = ibuf[p][0, pl.ds(off, L)]
                    obuf[p][0, pl.ds(off, L)] = plsc.load_gather(trow, [z16, v])
                out_store(p, s, d)

                @pl.when(s + 2 < seq)
                def _():
                    idx_load(p, s + 2)

            def pair(g, c):
                do_row(0, 2 * g, g == 0)
                do_row(1, 2 * g + 1, g == 0)
                return c

            lax.fori_loop(0, seq // 2, pair, 0)
            out_wait(0)
            out_wait(1)
            return carry

        lax.fori_loop(0, n_rounds, run_round, 0)

    return gather_kernel


def kernel(card_indices, table):
    batch, seq = card_indices.shape
    vocab, dim = table.shape
    idx_t = card_indices.astype(jnp.int32).T
    table_t = table.T
    gather = _make_gather(batch, seq, vocab, dim)
    out_t = gather(idx_t, table_t)
    return jnp.transpose(out_t, (2, 0, 1))


# speedup vs baseline: 2.1182x; 1.0013x over previous
import functools

import jax
import jax.numpy as jnp
from jax import lax
from jax.experimental import pallas as pl
from jax.experimental.pallas import tpu as pltpu
from jax.experimental.pallas import tpu_sc as plsc


@functools.cache
def _make_gather(batch, seq, V, D):
    info = plsc.get_sparse_core_info()
    L = info.num_lanes
    num_workers = info.num_cores * info.num_subcores
    n_rounds = D // num_workers
    unroll = 16
    n_groups = batch // (L * unroll)
    assert n_rounds * num_workers == D
    assert n_groups * L * unroll == batch and seq % 2 == 0

    mesh = plsc.VectorSubcoreMesh(core_axis_name="c", subcore_axis_name="s")

    @functools.partial(
        pl.kernel,
        mesh=mesh,
        out_type=jax.ShapeDtypeStruct((seq, D, batch), jnp.float32),
        scratch_types=[
            pltpu.VMEM((1, V), jnp.float32),
            *[pltpu.VMEM((1, batch), jnp.int32) for _ in range(2)],
            *[pltpu.VMEM((1, batch), jnp.float32) for _ in range(2)],
            *[pltpu.SemaphoreType.DMA for _ in range(2)],
            *[pltpu.SemaphoreType.DMA for _ in range(2)],
        ],
        compiler_params=pltpu.CompilerParams(
            use_tc_tiling_on_sc=True, needs_layout_passes=False
        ),
    )
    def gather_kernel(idx_hbm, table_hbm, out_hbm, trow, i0, i1, o0, o1, si0, si1, so0, so1):
        ibuf, obuf = (i0, i1), (o0, o1)
        isem, osem = (si0, si1), (so0, so1)
        wid = lax.axis_index("s") * info.num_cores + lax.axis_index("c")
        z16 = jnp.zeros((L,), jnp.int32)

        def idx_load(p, s):
            pltpu.async_copy(idx_hbm.at[pl.ds(s, 1)], ibuf[p], isem[p])

        def idx_wait(p):
            pltpu.make_async_copy(idx_hbm.at[pl.ds(0, 1)], ibuf[p], isem[p]).wait()

        def out_store(p, s, d):
            pltpu.async_copy(obuf[p], out_hbm.at[s, pl.ds(d, 1)], osem[p])

        def out_wait(p):
            pltpu.make_async_copy(obuf[p], out_hbm.at[0, pl.ds(0, 1)], osem[p]).wait()

        def run_round(r, carry):
            d = r * num_workers + wid
            pltpu.sync_copy(table_hbm.at[pl.ds(d, 1)], trow)
            idx_load(0, 0)
            idx_load(1, 1)

            def do_row(p, s, first_pair):
                idx_wait(p)

                @pl.when(jnp.logical_not(first_pair))
                def _():
                    out_wait(p)

                @plsc.parallel_loop(0, batch, L, unroll=unroll)
                def _inner(off):
                    v = ibuf[p][0, pl.ds(off, L)]
                    obuf[p][0, pl.ds(off, L)] = plsc.load_gather(trow, [z16, v])
                out_store(p, s, d)

                @pl.when(s + 2 < seq)
                def _():
                    idx_load(p, s + 2)

            def pair(g, c):
                do_row(0, 2 * g, g == 0)
                do_row(1, 2 * g + 1, g == 0)
                return c

            lax.fori_loop(0, seq // 2, pair, 0)
            out_wait(0)
            out_wait(1)
            return carry

        lax.fori_loop(0, n_rounds, run_round, 0)

    return gather_kernel


def kernel(card_indices, table):
    batch, seq = card_indices.shape
    vocab, dim = table.shape
    idx_t = card_indices.astype(jnp.int32).T
    table_t = table.T
    gather = _make_gather(batch, seq, vocab, dim)
    out_t = gather(idx_t, table_t)
    return jnp.transpose(out_t, (2, 0, 1))


# speedup vs baseline: 2.1195x; 1.0006x over previous
import functools

import jax
import jax.numpy as jnp
from jax import lax
from jax.experimental import pallas as pl
from jax.experimental.pallas import tpu as pltpu
from jax.experimental.pallas import tpu_sc as plsc

_NI = 2
_NO = 4


@functools.cache
def _make_gather(batch, seq, V, D):
    info = plsc.get_sparse_core_info()
    L = info.num_lanes
    num_workers = info.num_cores * info.num_subcores
    n_rounds = D // num_workers
    unroll = 16
    assert n_rounds * num_workers == D
    assert batch % (L * unroll) == 0 and seq % (_NI * _NO) == 0

    mesh = plsc.VectorSubcoreMesh(core_axis_name="c", subcore_axis_name="s")

    @functools.partial(
        pl.kernel,
        mesh=mesh,
        out_type=jax.ShapeDtypeStruct((seq, D, batch), jnp.float32),
        scratch_types=[
            pltpu.VMEM((1, V), jnp.float32),
            *[pltpu.VMEM((1, batch), jnp.int32) for _ in range(_NI)],
            *[pltpu.VMEM((1, batch), jnp.float32) for _ in range(_NO)],
            *[pltpu.SemaphoreType.DMA for _ in range(_NI + _NO)],
        ],
        compiler_params=pltpu.CompilerParams(
            use_tc_tiling_on_sc=True, needs_layout_passes=False
        ),
    )
    def gather_kernel(idx_hbm, table_hbm, out_hbm, trow, *bufs):
        ibuf = bufs[:_NI]
        obuf = bufs[_NI : _NI + _NO]
        isem = bufs[_NI + _NO : 2 * _NI + _NO]
        osem = bufs[2 * _NI + _NO :]
        wid = lax.axis_index("s") * info.num_cores + lax.axis_index("c")
        z16 = jnp.zeros((L,), jnp.int32)

        def idx_load(p, s):
            pltpu.async_copy(idx_hbm.at[pl.ds(s, 1)], ibuf[p], isem[p])

        def idx_wait(p):
            pltpu.make_async_copy(idx_hbm.at[pl.ds(0, 1)], ibuf[p], isem[p]).wait()

        def out_store(p, s, d):
            pltpu.async_copy(obuf[p], out_hbm.at[s, pl.ds(d, 1)], osem[p])

        def out_wait(p):
            pltpu.make_async_copy(obuf[p], out_hbm.at[0, pl.ds(0, 1)], osem[p]).wait()

        def run_round(r, carry):
            d = r * num_workers + wid
            pltpu.sync_copy(table_hbm.at[pl.ds(d, 1)], trow)
            for p in range(_NI):
                idx_load(p, p)

            def do_row(ip, op, s, first_use):
                idx_wait(ip)

                @pl.when(jnp.logical_not(first_use))
                def _():
                    out_wait(op)

                @plsc.parallel_loop(0, batch, L, unroll=unroll)
                def _inner(off):
                    v = ibuf[ip][0, pl.ds(off, L)]
                    obuf[op][0, pl.ds(off, L)] = plsc.load_gather(trow, [z16, v])

                out_store(op, s, d)

                @pl.when(s + _NI < seq)
                def _():
                    idx_load(ip, s + _NI)

            block = _NI * _NO

            def blk(g, c):
                for j in range(block):
                    first = (g == 0) if j < _NO else jnp.bool_(False)
                    do_row(j % _NI, j % _NO, g * block + j, first)
                return c

            lax.fori_loop(0, seq // block, blk, 0)
            for p in range(_NO):
                out_wait(p)
            return carry

        lax.fori_loop(0, n_rounds, run_round, 0)

    return gather_kernel


def kernel(card_indices, table):
    batch, seq = card_indices.shape
    vocab, dim = table.shape
    idx_t = card_indices.astype(jnp.int32).T
    table_t = table.T
    gather = _make_gather(batch, seq, vocab, dim)
    out_t = gather(idx_t, table_t)
    return jnp.transpose(out_t, (2, 0, 1))
